# baseline (speedup 1.0000x reference)
import functools

import jax
import jax.numpy as jnp
from jax import lax
from jax.experimental import pallas as pl
from jax.experimental.pallas import tpu as pltpu
from jax.experimental.pallas import tpu_sc as plsc

N = 10000
E = 320000
D_IN = 128
D_HID = 64
D_OUT = 5
D2 = 8

NC = 2
NS = 16
NW = NC * NS
L = 16

TRASH = N
N_PAD = 10240
B = 512
GRID = N_PAD // B

CHUNK = 128
CHUNKS_PER_TILE = 80
E_PAD = NW * CHUNKS_PER_TILE * CHUNK

_mesh = plsc.VectorSubcoreMesh(core_axis_name="c", subcore_axis_name="s")
_sc_params = pltpu.CompilerParams(
    needs_layout_passes=False, use_tc_tiling_on_sc=False
)

_EPT_H = E_PAD // NW


@functools.partial(
    pl.kernel,
    mesh=_mesh,
    out_type=jax.ShapeDtypeStruct((NW, N_PAD), jnp.float32),
    compiler_params=_sc_params,
    scratch_types=[
        pltpu.VMEM((_EPT_H,), jnp.int32),
        pltpu.VMEM((N_PAD,), jnp.float32),
    ],
)
def _hist_sc(dst_hbm, out_hbm, dst_v, hist_v):
    cid = lax.axis_index("c")
    sid = lax.axis_index("s")
    wid = cid * NS + sid
    pltpu.sync_copy(dst_hbm.at[pl.ds(wid * _EPT_H, _EPT_H)], dst_v)

    def zero(i, carry):
        hist_v[pl.ds(i * L, L)] = jnp.zeros((L,), jnp.float32)
        return carry

    lax.fori_loop(0, N_PAD // L, zero, 0)

    ones = jnp.ones((L,), jnp.float32)

    def body(i, carry):
        idx = dst_v[pl.ds(i * L, L)]
        plsc.addupdate_scatter(hist_v, [idx], ones)
        return carry

    lax.fori_loop(0, _EPT_H // L, body, 0)
    pltpu.sync_copy(hist_v, out_hbm.at[wid])


def _make_msgpass(D):
    NB = CHUNKS_PER_TILE
    STRIPE = N_PAD // NS

    @functools.partial(
        pl.kernel,
        mesh=_mesh,
        out_type=[
            jax.ShapeDtypeStruct((N_PAD, D), jnp.float32),
            jax.ShapeDtypeStruct((N_PAD, D), jnp.float32),
        ],
        compiler_params=_sc_params,
        scratch_types=[
            pltpu.VMEM((NB, CHUNK), jnp.int32),
            pltpu.VMEM((NB, CHUNK), jnp.int32),
            pltpu.VMEM((CHUNK, D), jnp.float32),
            pltpu.VMEM((CHUNK, D), jnp.float32),
            pltpu.VMEM((CHUNK, D), jnp.float32),
            pltpu.VMEM((CHUNK, D), jnp.float32),
            pltpu.VMEM_SHARED((N_PAD, D), jnp.float32),
            pltpu.SemaphoreType.DMA,
            pltpu.SemaphoreType.DMA,
        ],
    )
    def msgpass(src_hbm, dst_hbm, table_hbm, zeros_hbm, out0_hbm, out1_hbm,
                sidx_v, didx_v, r0, r1, r2, r3, acc_sh, gsem, ssem):
        cid = lax.axis_index("c")
        sid = lax.axis_index("s")
        bufs = [r0, r1, r2, r3]
        row0 = sid * STRIPE
        pltpu.sync_copy(zeros_hbm.at[pl.ds(row0, STRIPE)],
                        acc_sh.at[pl.ds(row0, STRIPE)])

        chunk0 = (cid * NS + sid) * NB
        pltpu.sync_copy(src_hbm.at[pl.ds(chunk0, NB)], sidx_v)
        pltpu.sync_copy(dst_hbm.at[pl.ds(chunk0, NB)], didx_v)
        plsc.subcore_barrier()

        pltpu.async_copy(table_hbm.at[sidx_v.at[0]], r0, gsem)
        pltpu.async_copy(table_hbm.at[sidx_v.at[1]], r1, gsem)

        def body(i, carry):
            j0 = i * 4
            for k in range(4):
                j = j0 + k
                buf = bufs[k]
                nbuf = bufs[(k + 2) % 4]
                pltpu.make_async_copy(
                    table_hbm.at[sidx_v.at[j]], buf, gsem).wait()

                @pl.when(j >= 2)
                def _():
                    pltpu.make_async_copy(
                        nbuf, acc_sh.at[didx_v.at[j - 2]], ssem).wait()

                @pl.when(j + 2 < NB)
                def _():
                    pltpu.async_copy(
                        table_hbm.at[sidx_v.at[j + 2]], nbuf, gsem)

                pltpu.async_copy(buf, acc_sh.at[didx_v.at[j]], ssem,
                                 add=True)
            return carry

        lax.fori_loop(0, NB // 4, body, 0)
        pltpu.make_async_copy(r2, acc_sh.at[didx_v.at[NB - 2]], ssem).wait()
        pltpu.make_async_copy(r3, acc_sh.at[didx_v.at[NB - 1]], ssem).wait()
        plsc.subcore_barrier()

        @pl.when(cid == 0)
        def _():
            pltpu.sync_copy(acc_sh.at[pl.ds(row0, STRIPE)],
                            out0_hbm.at[pl.ds(row0, STRIPE)])

        @pl.when(cid == 1)
        def _():
            pltpu.sync_copy(acc_sh.at[pl.ds(row0, STRIPE)],
                            out1_hbm.at[pl.ds(row0, STRIPE)])

    return msgpass


_msg8 = _make_msgpass(D2)

DC = D_HID // 2
NB_ALL = E_PAD // (NS * CHUNK)


@functools.partial(
    pl.kernel,
    mesh=_mesh,
    out_type=[
        jax.ShapeDtypeStruct((N_PAD, DC), jnp.float32),
        jax.ShapeDtypeStruct((N_PAD, DC), jnp.float32),
    ],
    compiler_params=_sc_params,
    scratch_types=[
        pltpu.VMEM((NB_ALL, CHUNK), jnp.int32),
        pltpu.VMEM((NB_ALL, CHUNK), jnp.int32),
        pltpu.VMEM((CHUNK, DC), jnp.float32),
        pltpu.VMEM((CHUNK, DC), jnp.float32),
        pltpu.VMEM((CHUNK, DC), jnp.float32),
        pltpu.VMEM((CHUNK, DC), jnp.float32),
        pltpu.VMEM_SHARED((N_PAD, DC), jnp.float32),
        pltpu.SemaphoreType.DMA,
        pltpu.SemaphoreType.DMA,
    ],
)
def _msg64c(src_hbm, dst_hbm, t0_hbm, t1_hbm, zeros_hbm, out0_hbm, out1_hbm,
            sidx_v, didx_v, r0, r1, r2, r3, acc_sh, gsem, ssem):
    cid = lax.axis_index("c")
    sid = lax.axis_index("s")
    bufs = [r0, r1, r2, r3]
    STRIPE = N_PAD // NS
    row0 = sid * STRIPE
    pltpu.sync_copy(zeros_hbm.at[pl.ds(row0, STRIPE)],
                    acc_sh.at[pl.ds(row0, STRIPE)])

    chunk0 = sid * NB_ALL
    pltpu.sync_copy(src_hbm.at[pl.ds(chunk0, NB_ALL)], sidx_v)
    pltpu.sync_copy(dst_hbm.at[pl.ds(chunk0, NB_ALL)], didx_v)
    plsc.subcore_barrier()

    def run(table_hbm, out_hbm):
        pltpu.async_copy(table_hbm.at[sidx_v.at[0]], r0, gsem)
        pltpu.async_copy(table_hbm.at[sidx_v.at[1]], r1, gsem)

        def body(i, carry):
            j0 = i * 4
            for k in range(4):
                j = j0 + k
                buf = bufs[k]
                nbuf = bufs[(k + 2) % 4]
                pltpu.make_async_copy(
                    table_hbm.at[sidx_v.at[j]], buf, gsem).wait()

                @pl.when(j >= 2)
                def _():
                    pltpu.make_async_copy(
                        nbuf, acc_sh.at[didx_v.at[j - 2]], ssem).wait()

                @pl.when(j + 2 < NB_ALL)
                def _():
                    pltpu.async_copy(
                        table_hbm.at[sidx_v.at[j + 2]], nbuf, gsem)

                pltpu.async_copy(buf, acc_sh.at[didx_v.at[j]], ssem,
                                 add=True)
            return carry

        lax.fori_loop(0, NB_ALL // 4, body, 0)
        pltpu.make_async_copy(
            r2, acc_sh.at[didx_v.at[NB_ALL - 2]], ssem).wait()
        pltpu.make_async_copy(
            r3, acc_sh.at[didx_v.at[NB_ALL - 1]], ssem).wait()
        plsc.subcore_barrier()
        pltpu.sync_copy(acc_sh.at[pl.ds(row0, STRIPE)],
                        out_hbm.at[pl.ds(row0, STRIPE)])

    @pl.when(cid == 0)
    def _():
        run(t0_hbm, out0_hbm)

    @pl.when(cid == 1)
    def _():
        run(t1_hbm, out1_hbm)


def _tc1_body(hist_ref, x_ref, w1_ref, hsa_ref, hsb_ref, dinv_ref):
    deg = 1.0 + jnp.sum(hist_ref[...], axis=0)
    dinv = lax.rsqrt(deg)
    h = jnp.dot(x_ref[...], w1_ref[...], preferred_element_type=jnp.float32)
    hs = h * dinv[:, None]
    hsa_ref[...] = hs[:, :DC]
    hsb_ref[...] = hs[:, DC:]
    dinv_ref[...] = dinv


_tc1 = pl.pallas_call(
    _tc1_body,
    grid=(GRID,),
    in_specs=[
        pl.BlockSpec((NW, B), lambda i: (0, i)),
        pl.BlockSpec((B, D_IN), lambda i: (i, 0)),
        pl.BlockSpec((D_IN, D_HID), lambda i: (0, 0)),
    ],
    out_specs=[
        pl.BlockSpec((B, DC), lambda i: (i, 0)),
        pl.BlockSpec((B, DC), lambda i: (i, 0)),
        pl.BlockSpec((B,), lambda i: (i,)),
    ],
    out_shape=[
        jax.ShapeDtypeStruct((N_PAD, DC), jnp.float32),
        jax.ShapeDtypeStruct((N_PAD, DC), jnp.float32),
        jax.ShapeDtypeStruct((N_PAD,), jnp.float32),
    ],
)


def _tc2_body(a0_ref, a1_ref, hsa_ref, hsb_ref, dinv_ref, w2_ref, b1_ref,
              hs2_ref):
    dinv = dinv_ref[...][:, None]
    s = jnp.concatenate(
        [a0_ref[...] + hsa_ref[...], a1_ref[...] + hsb_ref[...]], axis=1)
    out1 = dinv * s + b1_ref[...]
    r = jnp.maximum(out1, 0.0)
    h2 = jnp.dot(r, w2_ref[...], preferred_element_type=jnp.float32)
    hs2_ref[...] = h2 * dinv


_tc2 = pl.pallas_call(
    _tc2_body,
    grid=(GRID,),
    in_specs=[
        pl.BlockSpec((B, DC), lambda i: (i, 0)),
        pl.BlockSpec((B, DC), lambda i: (i, 0)),
        pl.BlockSpec((B, DC), lambda i: (i, 0)),
        pl.BlockSpec((B, DC), lambda i: (i, 0)),
        pl.BlockSpec((B,), lambda i: (i,)),
        pl.BlockSpec((D_HID, D2), lambda i: (0, 0)),
        pl.BlockSpec((1, D_HID), lambda i: (0, 0)),
    ],
    out_specs=pl.BlockSpec((B, D2), lambda i: (i, 0)),
    out_shape=jax.ShapeDtypeStruct((N_PAD, D2), jnp.float32),
)


def _tc3_body(a0_ref, a1_ref, hs2_ref, dinv_ref, b2_ref, out_ref):
    dinv = dinv_ref[...][:, None]
    out_ref[...] = dinv * (a0_ref[...] + a1_ref[...] + hs2_ref[...]) + b2_ref[...]


_tc3 = pl.pallas_call(
    _tc3_body,
    grid=(GRID,),
    in_specs=[
        pl.BlockSpec((B, D2), lambda i: (i, 0)),
        pl.BlockSpec((B, D2), lambda i: (i, 0)),
        pl.BlockSpec((B, D2), lambda i: (i, 0)),
        pl.BlockSpec((B,), lambda i: (i,)),
        pl.BlockSpec((1, D2), lambda i: (0, 0)),
    ],
    out_specs=pl.BlockSpec((B, D2), lambda i: (i, 0)),
    out_shape=jax.ShapeDtypeStruct((N_PAD, D2), jnp.float32),
)


def kernel(x, edge_index, W1, b1, W2, b2):
    src = edge_index[0].astype(jnp.int32)
    dst = edge_index[1].astype(jnp.int32)
    pad = E_PAD - E
    src_p = jnp.concatenate([src, jnp.full((pad,), TRASH, jnp.int32)])
    dst_p = jnp.concatenate([dst, jnp.full((pad,), TRASH, jnp.int32)])
    x_p = jnp.pad(x, ((0, N_PAD - N), (0, 0)))
    w2_p = jnp.pad(W2, ((0, 0), (0, D2 - D_OUT)))
    b1_2d = b1.reshape(1, D_HID)
    b2_2d = jnp.pad(b2.reshape(1, D_OUT), ((0, 0), (0, D2 - D_OUT)))

    src_2d = src_p.reshape(NW * CHUNKS_PER_TILE, CHUNK)
    dst_2d = dst_p.reshape(NW * CHUNKS_PER_TILE, CHUNK)

    hist = _hist_sc(dst_p)
    hsa, hsb, dinv = _tc1(hist, x_p, W1)
    z32 = jnp.zeros((N_PAD, DC), jnp.float32)
    a0, a1 = _msg64c(src_2d, dst_2d, hsa, hsb, z32)
    hs2 = _tc2(a0, a1, hsa, hsb, dinv, w2_p, b1_2d)
    z8 = jnp.zeros((N_PAD, D2), jnp.float32)
    c0, c1 = _msg8(src_2d, dst_2d, hs2, z8)
    out = _tc3(c0, c1, hs2, dinv, b2_2d)
    return out[:N, :D_OUT]


# speedup vs baseline: 1.6369x; 1.6369x over previous
import functools

import jax
import jax.numpy as jnp
from jax import lax
from jax.experimental import pallas as pl
from jax.experimental.pallas import tpu as pltpu
from jax.experimental.pallas import tpu_sc as plsc

N = 10000
E = 320000
D_IN = 128
D_HID = 64
D_OUT = 5
D2 = 8

NC = 2
NS = 16
NW = NC * NS
L = 16

TRASH = N
N_PAD = 10240
B = 512
GRID = N_PAD // B

CHUNK = 128
CHUNKS_PER_TILE = 80
E_PAD = NW * CHUNKS_PER_TILE * CHUNK

_mesh = plsc.VectorSubcoreMesh(core_axis_name="c", subcore_axis_name="s")
_sc_params = pltpu.CompilerParams(
    needs_layout_passes=False, use_tc_tiling_on_sc=False
)

_EPT_H = E_PAD // NW


@functools.partial(
    pl.kernel,
    mesh=_mesh,
    out_type=jax.ShapeDtypeStruct((NW, N_PAD), jnp.float32),
    compiler_params=_sc_params,
    scratch_types=[
        pltpu.VMEM((_EPT_H,), jnp.int32),
        pltpu.VMEM((N_PAD,), jnp.float32),
    ],
)
def _hist_sc(dst_hbm, out_hbm, dst_v, hist_v):
    cid = lax.axis_index("c")
    sid = lax.axis_index("s")
    wid = cid * NS + sid
    pltpu.sync_copy(dst_hbm.at[pl.ds(wid * _EPT_H, _EPT_H)], dst_v)

    def zero(i, carry):
        hist_v[pl.ds(i * L, L)] = jnp.zeros((L,), jnp.float32)
        return carry

    lax.fori_loop(0, N_PAD // L, zero, 0)

    ones = jnp.ones((L,), jnp.float32)

    def body(i, carry):
        idx = dst_v[pl.ds(i * L, L)]
        plsc.addupdate_scatter(hist_v, [idx], ones)
        return carry

    lax.fori_loop(0, _EPT_H // L, body, 0)
    pltpu.sync_copy(hist_v, out_hbm.at[wid])


def _make_msgpass(D):
    NB = CHUNKS_PER_TILE
    STRIPE = N_PAD // NS

    @functools.partial(
        pl.kernel,
        mesh=_mesh,
        out_type=[
            jax.ShapeDtypeStruct((N_PAD, D), jnp.float32),
            jax.ShapeDtypeStruct((N_PAD, D), jnp.float32),
        ],
        compiler_params=_sc_params,
        scratch_types=[
            pltpu.VMEM((NB, CHUNK), jnp.int32),
            pltpu.VMEM((NB, CHUNK), jnp.int32),
            pltpu.VMEM((CHUNK, D), jnp.float32),
            pltpu.VMEM((CHUNK, D), jnp.float32),
            pltpu.VMEM((CHUNK, D), jnp.float32),
            pltpu.VMEM((CHUNK, D), jnp.float32),
            pltpu.VMEM_SHARED((N_PAD, D), jnp.float32),
            pltpu.VMEM_SHARED((N_PAD, D), jnp.float32),
            pltpu.SemaphoreType.DMA,
            pltpu.SemaphoreType.DMA,
        ],
    )
    def msgpass(src_hbm, dst_hbm, table_hbm, zeros_hbm, out0_hbm, out1_hbm,
                sidx_v, didx_v, r0, r1, r2, r3, acc_sh, table_sh, gsem, ssem):
        cid = lax.axis_index("c")
        sid = lax.axis_index("s")
        bufs = [r0, r1, r2, r3]
        row0 = sid * STRIPE
        pltpu.sync_copy(zeros_hbm.at[pl.ds(row0, STRIPE)],
                        acc_sh.at[pl.ds(row0, STRIPE)])
        pltpu.sync_copy(table_hbm.at[pl.ds(row0, STRIPE)],
                        table_sh.at[pl.ds(row0, STRIPE)])

        chunk0 = (cid * NS + sid) * NB
        pltpu.sync_copy(src_hbm.at[pl.ds(chunk0, NB)], sidx_v)
        pltpu.sync_copy(dst_hbm.at[pl.ds(chunk0, NB)], didx_v)
        plsc.subcore_barrier()

        pltpu.async_copy(table_sh.at[sidx_v.at[0]], r0, gsem)
        pltpu.async_copy(table_sh.at[sidx_v.at[1]], r1, gsem)

        def body(i, carry):
            j0 = i * 4
            for k in range(4):
                j = j0 + k
                buf = bufs[k]
                nbuf = bufs[(k + 2) % 4]
                pltpu.make_async_copy(
                    table_sh.at[sidx_v.at[j]], buf, gsem).wait()

                @pl.when(j >= 2)
                def _():
                    pltpu.make_async_copy(
                        nbuf, acc_sh.at[didx_v.at[j - 2]], ssem).wait()

                @pl.when(j + 2 < NB)
                def _():
                    pltpu.async_copy(
                        table_sh.at[sidx_v.at[j + 2]], nbuf, gsem)

                pltpu.async_copy(buf, acc_sh.at[didx_v.at[j]], ssem,
                                 add=True)
            return carry

        lax.fori_loop(0, NB // 4, body, 0)
        pltpu.make_async_copy(r2, acc_sh.at[didx_v.at[NB - 2]], ssem).wait()
        pltpu.make_async_copy(r3, acc_sh.at[didx_v.at[NB - 1]], ssem).wait()
        plsc.subcore_barrier()

        @pl.when(cid == 0)
        def _():
            pltpu.sync_copy(acc_sh.at[pl.ds(row0, STRIPE)],
                            out0_hbm.at[pl.ds(row0, STRIPE)])

        @pl.when(cid == 1)
        def _():
            pltpu.sync_copy(acc_sh.at[pl.ds(row0, STRIPE)],
                            out1_hbm.at[pl.ds(row0, STRIPE)])

    return msgpass


_msg8 = _make_msgpass(D2)

DC = D_HID // 2
NB_ALL = E_PAD // (NS * CHUNK)


@functools.partial(
    pl.kernel,
    mesh=_mesh,
    out_type=[
        jax.ShapeDtypeStruct((N_PAD, DC), jnp.float32),
        jax.ShapeDtypeStruct((N_PAD, DC), jnp.float32),
    ],
    compiler_params=_sc_params,
    scratch_types=[
        pltpu.VMEM((NB_ALL, CHUNK), jnp.int32),
        pltpu.VMEM((NB_ALL, CHUNK), jnp.int32),
        pltpu.VMEM((CHUNK, DC), jnp.float32),
        pltpu.VMEM((CHUNK, DC), jnp.float32),
        pltpu.VMEM((CHUNK, DC), jnp.float32),
        pltpu.VMEM((CHUNK, DC), jnp.float32),
        pltpu.VMEM_SHARED((N_PAD, DC), jnp.float32),
        pltpu.VMEM_SHARED((N_PAD, DC), jnp.float32),
        pltpu.SemaphoreType.DMA,
        pltpu.SemaphoreType.DMA,
    ],
)
def _msg64c(src_hbm, dst_hbm, t0_hbm, t1_hbm, zeros_hbm, out0_hbm, out1_hbm,
            sidx_v, didx_v, r0, r1, r2, r3, acc_sh, table_sh, gsem, ssem):
    cid = lax.axis_index("c")
    sid = lax.axis_index("s")
    bufs = [r0, r1, r2, r3]
    STRIPE = N_PAD // NS
    row0 = sid * STRIPE
    pltpu.sync_copy(zeros_hbm.at[pl.ds(row0, STRIPE)],
                    acc_sh.at[pl.ds(row0, STRIPE)])

    @pl.when(cid == 0)
    def _():
        pltpu.sync_copy(t0_hbm.at[pl.ds(row0, STRIPE)],
                        table_sh.at[pl.ds(row0, STRIPE)])

    @pl.when(cid == 1)
    def _():
        pltpu.sync_copy(t1_hbm.at[pl.ds(row0, STRIPE)],
                        table_sh.at[pl.ds(row0, STRIPE)])

    chunk0 = sid * NB_ALL
    pltpu.sync_copy(src_hbm.at[pl.ds(chunk0, NB_ALL)], sidx_v)
    pltpu.sync_copy(dst_hbm.at[pl.ds(chunk0, NB_ALL)], didx_v)
    plsc.subcore_barrier()

    def run(table_hbm, out_hbm):
        del table_hbm
        pltpu.async_copy(table_sh.at[sidx_v.at[0]], r0, gsem)
        pltpu.async_copy(table_sh.at[sidx_v.at[1]], r1, gsem)

        def body(i, carry):
            j0 = i * 4
            for k in range(4):
                j = j0 + k
                buf = bufs[k]
                nbuf = bufs[(k + 2) % 4]
                pltpu.make_async_copy(
                    table_sh.at[sidx_v.at[j]], buf, gsem).wait()

                @pl.when(j >= 2)
                def _():
                    pltpu.make_async_copy(
                        nbuf, acc_sh.at[didx_v.at[j - 2]], ssem).wait()

                @pl.when(j + 2 < NB_ALL)
                def _():
                    pltpu.async_copy(
                        table_sh.at[sidx_v.at[j + 2]], nbuf, gsem)

                pltpu.async_copy(buf, acc_sh.at[didx_v.at[j]], ssem,
                                 add=True)
            return carry

        lax.fori_loop(0, NB_ALL // 4, body, 0)
        pltpu.make_async_copy(
            r2, acc_sh.at[didx_v.at[NB_ALL - 2]], ssem).wait()
        pltpu.make_async_copy(
            r3, acc_sh.at[didx_v.at[NB_ALL - 1]], ssem).wait()
        plsc.subcore_barrier()
        pltpu.sync_copy(acc_sh.at[pl.ds(row0, STRIPE)],
                        out_hbm.at[pl.ds(row0, STRIPE)])

    @pl.when(cid == 0)
    def _():
        run(t0_hbm, out0_hbm)

    @pl.when(cid == 1)
    def _():
        run(t1_hbm, out1_hbm)


def _tc1_body(hist_ref, x_ref, w1_ref, hsa_ref, hsb_ref, dinv_ref):
    deg = 1.0 + jnp.sum(hist_ref[...], axis=0)
    dinv = lax.rsqrt(deg)
    h = jnp.dot(x_ref[...], w1_ref[...], preferred_element_type=jnp.float32)
    hs = h * dinv[:, None]
    hsa_ref[...] = hs[:, :DC]
    hsb_ref[...] = hs[:, DC:]
    dinv_ref[...] = dinv


_tc1 = pl.pallas_call(
    _tc1_body,
    grid=(GRID,),
    in_specs=[
        pl.BlockSpec((NW, B), lambda i: (0, i)),
        pl.BlockSpec((B, D_IN), lambda i: (i, 0)),
        pl.BlockSpec((D_IN, D_HID), lambda i: (0, 0)),
    ],
    out_specs=[
        pl.BlockSpec((B, DC), lambda i: (i, 0)),
        pl.BlockSpec((B, DC), lambda i: (i, 0)),
        pl.BlockSpec((B,), lambda i: (i,)),
    ],
    out_shape=[
        jax.ShapeDtypeStruct((N_PAD, DC), jnp.float32),
        jax.ShapeDtypeStruct((N_PAD, DC), jnp.float32),
        jax.ShapeDtypeStruct((N_PAD,), jnp.float32),
    ],
)


def _tc2_body(a0_ref, a1_ref, hsa_ref, hsb_ref, dinv_ref, w2_ref, b1_ref,
              hs2_ref):
    dinv = dinv_ref[...][:, None]
    s = jnp.concatenate(
        [a0_ref[...] + hsa_ref[...], a1_ref[...] + hsb_ref[...]], axis=1)
    out1 = dinv * s + b1_ref[...]
    r = jnp.maximum(out1, 0.0)
    h2 = jnp.dot(r, w2_ref[...], preferred_element_type=jnp.float32)
    hs2_ref[...] = h2 * dinv


_tc2 = pl.pallas_call(
    _tc2_body,
    grid=(GRID,),
    in_specs=[
        pl.BlockSpec((B, DC), lambda i: (i, 0)),
        pl.BlockSpec((B, DC), lambda i: (i, 0)),
        pl.BlockSpec((B, DC), lambda i: (i, 0)),
        pl.BlockSpec((B, DC), lambda i: (i, 0)),
        pl.BlockSpec((B,), lambda i: (i,)),
        pl.BlockSpec((D_HID, D2), lambda i: (0, 0)),
        pl.BlockSpec((1, D_HID), lambda i: (0, 0)),
    ],
    out_specs=pl.BlockSpec((B, D2), lambda i: (i, 0)),
    out_shape=jax.ShapeDtypeStruct((N_PAD, D2), jnp.float32),
)


def _tc3_body(a0_ref, a1_ref, hs2_ref, dinv_ref, b2_ref, out_ref):
    dinv = dinv_ref[...][:, None]
    out_ref[...] = dinv * (a0_ref[...] + a1_ref[...] + hs2_ref[...]) + b2_ref[...]


_tc3 = pl.pallas_call(
    _tc3_body,
    grid=(GRID,),
    in_specs=[
        pl.BlockSpec((B, D2), lambda i: (i, 0)),
        pl.BlockSpec((B, D2), lambda i: (i, 0)),
        pl.BlockSpec((B, D2), lambda i: (i, 0)),
        pl.BlockSpec((B,), lambda i: (i,)),
        pl.BlockSpec((1, D2), lambda i: (0, 0)),
    ],
    out_specs=pl.BlockSpec((B, D2), lambda i: (i, 0)),
    out_shape=jax.ShapeDtypeStruct((N_PAD, D2), jnp.float32),
)


def kernel(x, edge_index, W1, b1, W2, b2):
    src = edge_index[0].astype(jnp.int32)
    dst = edge_index[1].astype(jnp.int32)
    pad = E_PAD - E
    src_p = jnp.concatenate([src, jnp.full((pad,), TRASH, jnp.int32)])
    dst_p = jnp.concatenate([dst, jnp.full((pad,), TRASH, jnp.int32)])
    x_p = jnp.pad(x, ((0, N_PAD - N), (0, 0)))
    w2_p = jnp.pad(W2, ((0, 0), (0, D2 - D_OUT)))
    b1_2d = b1.reshape(1, D_HID)
    b2_2d = jnp.pad(b2.reshape(1, D_OUT), ((0, 0), (0, D2 - D_OUT)))

    src_2d = src_p.reshape(NW * CHUNKS_PER_TILE, CHUNK)
    dst_2d = dst_p.reshape(NW * CHUNKS_PER_TILE, CHUNK)

    hist = _hist_sc(dst_p)
    hsa, hsb, dinv = _tc1(hist, x_p, W1)
    z32 = jnp.zeros((N_PAD, DC), jnp.float32)
    a0, a1 = _msg64c(src_2d, dst_2d, hsa, hsb, z32)
    hs2 = _tc2(a0, a1, hsa, hsb, dinv, w2_p, b1_2d)
    z8 = jnp.zeros((N_PAD, D2), jnp.float32)
    c0, c1 = _msg8(src_2d, dst_2d, hs2, z8)
    out = _tc3(c0, c1, hs2, dinv, b2_2d)
    return out[:N, :D_OUT]


# speedup vs baseline: 1.6740x; 1.0227x over previous
import functools

import jax
import jax.numpy as jnp
from jax import lax
from jax.experimental import pallas as pl
from jax.experimental.pallas import tpu as pltpu
from jax.experimental.pallas import tpu_sc as plsc

N = 10000
E = 320000
D_IN = 128
D_HID = 64
D_OUT = 5
D2 = 8

NC = 2
NS = 16
NW = NC * NS
L = 16

TRASH = N
N_PAD = 10240
B = 512
GRID = N_PAD // B

CHUNK = 128
CHUNKS_PER_TILE = 80
E_PAD = NW * CHUNKS_PER_TILE * CHUNK

_mesh = plsc.VectorSubcoreMesh(core_axis_name="c", subcore_axis_name="s")
_sc_params = pltpu.CompilerParams(
    needs_layout_passes=False, use_tc_tiling_on_sc=False
)

_EPT_H = E_PAD // NW


@functools.partial(
    pl.kernel,
    mesh=_mesh,
    out_type=jax.ShapeDtypeStruct((NW, N_PAD), jnp.float32),
    compiler_params=_sc_params,
    scratch_types=[
        pltpu.VMEM((_EPT_H,), jnp.int32),
        pltpu.VMEM((N_PAD,), jnp.float32),
    ],
)
def _hist_sc(dst_hbm, out_hbm, dst_v, hist_v):
    cid = lax.axis_index("c")
    sid = lax.axis_index("s")
    wid = cid * NS + sid
    pltpu.sync_copy(dst_hbm.at[pl.ds(wid * _EPT_H, _EPT_H)], dst_v)

    def zero(i, carry):
        hist_v[pl.ds(i * L, L)] = jnp.zeros((L,), jnp.float32)
        return carry

    lax.fori_loop(0, N_PAD // L, zero, 0)

    ones = jnp.ones((L,), jnp.float32)

    def body(i, carry):
        idx = dst_v[pl.ds(i * L, L)]
        plsc.addupdate_scatter(hist_v, [idx], ones)
        return carry

    lax.fori_loop(0, _EPT_H // L, body, 0)
    pltpu.sync_copy(hist_v, out_hbm.at[wid])


def _make_msgpass(D):
    NB = CHUNKS_PER_TILE
    STRIPE = N_PAD // NS

    @functools.partial(
        pl.kernel,
        mesh=_mesh,
        out_type=[
            jax.ShapeDtypeStruct((N_PAD, D), jnp.float32),
            jax.ShapeDtypeStruct((N_PAD, D), jnp.float32),
        ],
        compiler_params=_sc_params,
        scratch_types=[
            pltpu.VMEM((NB, CHUNK), jnp.int32),
            pltpu.VMEM((NB, CHUNK), jnp.int32),
            pltpu.VMEM((CHUNK, D), jnp.float32),
            pltpu.VMEM((CHUNK, D), jnp.float32),
            pltpu.VMEM((CHUNK, D), jnp.float32),
            pltpu.VMEM((CHUNK, D), jnp.float32),
            pltpu.VMEM_SHARED((N_PAD, D), jnp.float32),
            pltpu.VMEM_SHARED((N_PAD, D), jnp.float32),
            pltpu.SemaphoreType.DMA,
            pltpu.SemaphoreType.DMA,
        ],
    )
    def msgpass(src_hbm, dst_hbm, table_hbm, zeros_hbm, out0_hbm, out1_hbm,
                sidx_v, didx_v, r0, r1, r2, r3, acc_sh, table_sh, gsem, ssem):
        cid = lax.axis_index("c")
        sid = lax.axis_index("s")
        bufs = [r0, r1, r2, r3]
        row0 = sid * STRIPE
        chunk0 = (cid * NS + sid) * NB
        z = pltpu.async_copy(zeros_hbm.at[pl.ds(row0, STRIPE)],
                             acc_sh.at[pl.ds(row0, STRIPE)], ssem)
        t = pltpu.async_copy(table_hbm.at[pl.ds(row0, STRIPE)],
                             table_sh.at[pl.ds(row0, STRIPE)], ssem)
        s = pltpu.async_copy(src_hbm.at[pl.ds(chunk0, NB)], sidx_v, ssem)
        d = pltpu.async_copy(dst_hbm.at[pl.ds(chunk0, NB)], didx_v, ssem)
        z.wait()
        t.wait()
        s.wait()
        d.wait()
        plsc.subcore_barrier()

        pltpu.async_copy(table_sh.at[sidx_v.at[0]], r0, gsem)
        pltpu.async_copy(table_sh.at[sidx_v.at[1]], r1, gsem)

        def body(i, carry):
            j0 = i * 4
            for k in range(4):
                j = j0 + k
                buf = bufs[k]
                nbuf = bufs[(k + 2) % 4]
                pltpu.make_async_copy(
                    table_sh.at[sidx_v.at[j]], buf, gsem).wait()

                @pl.when(j >= 2)
                def _():
                    pltpu.make_async_copy(
                        nbuf, acc_sh.at[didx_v.at[j - 2]], ssem).wait()

                @pl.when(j + 2 < NB)
                def _():
                    pltpu.async_copy(
                        table_sh.at[sidx_v.at[j + 2]], nbuf, gsem)

                pltpu.async_copy(buf, acc_sh.at[didx_v.at[j]], ssem,
                                 add=True)
            return carry

        lax.fori_loop(0, NB // 4, body, 0)
        pltpu.make_async_copy(r2, acc_sh.at[didx_v.at[NB - 2]], ssem).wait()
        pltpu.make_async_copy(r3, acc_sh.at[didx_v.at[NB - 1]], ssem).wait()
        plsc.subcore_barrier()

        @pl.when(cid == 0)
        def _():
            pltpu.sync_copy(acc_sh.at[pl.ds(row0, STRIPE)],
                            out0_hbm.at[pl.ds(row0, STRIPE)])

        @pl.when(cid == 1)
        def _():
            pltpu.sync_copy(acc_sh.at[pl.ds(row0, STRIPE)],
                            out1_hbm.at[pl.ds(row0, STRIPE)])

    return msgpass


_msg8 = _make_msgpass(D2)

DC = D_HID // 2
NB_ALL = E_PAD // (NS * CHUNK)


@functools.partial(
    pl.kernel,
    mesh=_mesh,
    out_type=[
        jax.ShapeDtypeStruct((N_PAD, DC), jnp.float32),
        jax.ShapeDtypeStruct((N_PAD, DC), jnp.float32),
    ],
    compiler_params=_sc_params,
    scratch_types=[
        pltpu.VMEM((NB_ALL, CHUNK), jnp.int32),
        pltpu.VMEM((NB_ALL, CHUNK), jnp.int32),
        pltpu.VMEM((CHUNK, DC), jnp.float32),
        pltpu.VMEM((CHUNK, DC), jnp.float32),
        pltpu.VMEM((CHUNK, DC), jnp.float32),
        pltpu.VMEM((CHUNK, DC), jnp.float32),
        pltpu.VMEM_SHARED((N_PAD, DC), jnp.float32),
        pltpu.VMEM_SHARED((N_PAD, DC), jnp.float32),
        pltpu.SemaphoreType.DMA,
        pltpu.SemaphoreType.DMA,
    ],
)
def _msg64c(src_hbm, dst_hbm, t0_hbm, t1_hbm, zeros_hbm, out0_hbm, out1_hbm,
            sidx_v, didx_v, r0, r1, r2, r3, acc_sh, table_sh, gsem, ssem):
    cid = lax.axis_index("c")
    sid = lax.axis_index("s")
    bufs = [r0, r1, r2, r3]
    STRIPE = N_PAD // NS
    row0 = sid * STRIPE
    chunk0 = sid * NB_ALL

    z = pltpu.async_copy(zeros_hbm.at[pl.ds(row0, STRIPE)],
                         acc_sh.at[pl.ds(row0, STRIPE)], ssem)

    @pl.when(cid == 0)
    def _():
        pltpu.async_copy(t0_hbm.at[pl.ds(row0, STRIPE)],
                         table_sh.at[pl.ds(row0, STRIPE)], ssem)

    @pl.when(cid == 1)
    def _():
        pltpu.async_copy(t1_hbm.at[pl.ds(row0, STRIPE)],
                         table_sh.at[pl.ds(row0, STRIPE)], ssem)

    s = pltpu.async_copy(src_hbm.at[pl.ds(chunk0, NB_ALL)], sidx_v, ssem)
    d = pltpu.async_copy(dst_hbm.at[pl.ds(chunk0, NB_ALL)], didx_v, ssem)
    z.wait()
    pltpu.make_async_copy(t0_hbm.at[pl.ds(row0, STRIPE)],
                          table_sh.at[pl.ds(row0, STRIPE)], ssem).wait()
    s.wait()
    d.wait()
    plsc.subcore_barrier()

    def run(table_hbm, out_hbm):
        del table_hbm
        pltpu.async_copy(table_sh.at[sidx_v.at[0]], r0, gsem)
        pltpu.async_copy(table_sh.at[sidx_v.at[1]], r1, gsem)

        def body(i, carry):
            j0 = i * 4
            for k in range(4):
                j = j0 + k
                buf = bufs[k]
                nbuf = bufs[(k + 2) % 4]
                pltpu.make_async_copy(
                    table_sh.at[sidx_v.at[j]], buf, gsem).wait()

                @pl.when(j >= 2)
                def _():
                    pltpu.make_async_copy(
                        nbuf, acc_sh.at[didx_v.at[j - 2]], ssem).wait()

                @pl.when(j + 2 < NB_ALL)
                def _():
                    pltpu.async_copy(
                        table_sh.at[sidx_v.at[j + 2]], nbuf, gsem)

                pltpu.async_copy(buf, acc_sh.at[didx_v.at[j]], ssem,
                                 add=True)
            return carry

        lax.fori_loop(0, NB_ALL // 4, body, 0)
        pltpu.make_async_copy(
            r2, acc_sh.at[didx_v.at[NB_ALL - 2]], ssem).wait()
        pltpu.make_async_copy(
            r3, acc_sh.at[didx_v.at[NB_ALL - 1]], ssem).wait()
        plsc.subcore_barrier()
        pltpu.sync_copy(acc_sh.at[pl.ds(row0, STRIPE)],
                        out_hbm.at[pl.ds(row0, STRIPE)])

    @pl.when(cid == 0)
    def _():
        run(t0_hbm, out0_hbm)

    @pl.when(cid == 1)
    def _():
        run(t1_hbm, out1_hbm)


def _tc1_body(hist_ref, x_ref, w1_ref, hsa_ref, hsb_ref, dinv_ref):
    deg = 1.0 + jnp.sum(hist_ref[...], axis=0)
    dinv = lax.rsqrt(deg)
    h = jnp.dot(x_ref[...], w1_ref[...], preferred_element_type=jnp.float32)
    hs = h * dinv[:, None]
    hsa_ref[...] = hs[:, :DC]
    hsb_ref[...] = hs[:, DC:]
    dinv_ref[...] = dinv


_tc1 = pl.pallas_call(
    _tc1_body,
    grid=(GRID,),
    in_specs=[
        pl.BlockSpec((NW, B), lambda i: (0, i)),
        pl.BlockSpec((B, D_IN), lambda i: (i, 0)),
        pl.BlockSpec((D_IN, D_HID), lambda i: (0, 0)),
    ],
    out_specs=[
        pl.BlockSpec((B, DC), lambda i: (i, 0)),
        pl.BlockSpec((B, DC), lambda i: (i, 0)),
        pl.BlockSpec((B,), lambda i: (i,)),
    ],
    out_shape=[
        jax.ShapeDtypeStruct((N_PAD, DC), jnp.float32),
        jax.ShapeDtypeStruct((N_PAD, DC), jnp.float32),
        jax.ShapeDtypeStruct((N_PAD,), jnp.float32),
    ],
)


def _tc2_body(a0_ref, a1_ref, hsa_ref, hsb_ref, dinv_ref, w2_ref, b1_ref,
              hs2_ref):
    dinv = dinv_ref[...][:, None]
    s = jnp.concatenate(
        [a0_ref[...] + hsa_ref[...], a1_ref[...] + hsb_ref[...]], axis=1)
    out1 = dinv * s + b1_ref[...]
    r = jnp.maximum(out1, 0.0)
    h2 = jnp.dot(r, w2_ref[...], preferred_element_type=jnp.float32)
    hs2_ref[...] = h2 * dinv


_tc2 = pl.pallas_call(
    _tc2_body,
    grid=(GRID,),
    in_specs=[
        pl.BlockSpec((B, DC), lambda i: (i, 0)),
        pl.BlockSpec((B, DC), lambda i: (i, 0)),
        pl.BlockSpec((B, DC), lambda i: (i, 0)),
        pl.BlockSpec((B, DC), lambda i: (i, 0)),
        pl.BlockSpec((B,), lambda i: (i,)),
        pl.BlockSpec((D_HID, D2), lambda i: (0, 0)),
        pl.BlockSpec((1, D_HID), lambda i: (0, 0)),
    ],
    out_specs=pl.BlockSpec((B, D2), lambda i: (i, 0)),
    out_shape=jax.ShapeDtypeStruct((N_PAD, D2), jnp.float32),
)


def _tc3_body(a0_ref, a1_ref, hs2_ref, dinv_ref, b2_ref, out_ref):
    dinv = dinv_ref[...][:, None]
    out_ref[...] = dinv * (a0_ref[...] + a1_ref[...] + hs2_ref[...]) + b2_ref[...]


_tc3 = pl.pallas_call(
    _tc3_body,
    grid=(GRID,),
    in_specs=[
        pl.BlockSpec((B, D2), lambda i: (i, 0)),
        pl.BlockSpec((B, D2), lambda i: (i, 0)),
        pl.BlockSpec((B, D2), lambda i: (i, 0)),
        pl.BlockSpec((B,), lambda i: (i,)),
        pl.BlockSpec((1, D2), lambda i: (0, 0)),
    ],
    out_specs=pl.BlockSpec((B, D2), lambda i: (i, 0)),
    out_shape=jax.ShapeDtypeStruct((N_PAD, D2), jnp.float32),
)


def kernel(x, edge_index, W1, b1, W2, b2):
    src = edge_index[0].astype(jnp.int32)
    dst = edge_index[1].astype(jnp.int32)
    pad = E_PAD - E
    src_p = jnp.concatenate([src, jnp.full((pad,), TRASH, jnp.int32)])
    dst_p = jnp.concatenate([dst, jnp.full((pad,), TRASH, jnp.int32)])
    x_p = jnp.pad(x, ((0, N_PAD - N), (0, 0)))
    w2_p = jnp.pad(W2, ((0, 0), (0, D2 - D_OUT)))
    b1_2d = b1.reshape(1, D_HID)
    b2_2d = jnp.pad(b2.reshape(1, D_OUT), ((0, 0), (0, D2 - D_OUT)))

    src_2d = src_p.reshape(NW * CHUNKS_PER_TILE, CHUNK)
    dst_2d = dst_p.reshape(NW * CHUNKS_PER_TILE, CHUNK)

    hist = _hist_sc(dst_p)
    hsa, hsb, dinv = _tc1(hist, x_p, W1)
    z32 = jnp.zeros((N_PAD, DC), jnp.float32)
    a0, a1 = _msg64c(src_2d, dst_2d, hsa, hsb, z32)
    hs2 = _tc2(a0, a1, hsa, hsb, dinv, w2_p, b1_2d)
    z8 = jnp.zeros((N_PAD, D2), jnp.float32)
    c0, c1 = _msg8(src_2d, dst_2d, hs2, z8)
    out = _tc3(c0, c1, hs2, dinv, b2_2d)
    return out[:N, :D_OUT]
